# baseline (speedup 1.0000x reference)
import jax
import jax.numpy as jnp
from jax.experimental import pallas as pl
from jax.experimental.pallas import tpu as pltpu

_N = 1000
_NP = 1024
_D = 64
_IN = 16
_K = 20
_B = 32
_BLK = 128
_NB = _NP // _BLK
_AB = 8
_GB = 16
_CNT = float(_B * _N)
_EPS = 1e-5


def _attn_kernel(emb_ref, embb_ref, data_ref, lw_ref, atti_ref, attj_ref,
                 attemi_ref, attemj_ref, gb_ref, out_ref, s1_ref, s2_ref,
                 bias_scr):
    i = pl.program_id(0)

    @pl.when(i < _NB)
    def _mask_phase():
        w = emb_ref[...]
        wb = embb_ref[...]
        n_full = jnp.sqrt(jnp.maximum(jnp.sum(w * w, axis=1), 1e-12))
        n_blk = jnp.sqrt(jnp.maximum(jnp.sum(wb * wb, axis=1), 1e-12))
        cos = jax.lax.dot_general(
            wb, w, (((1,), (1,)), ((), ())),
            preferred_element_type=jnp.float32)
        cos = cos / (n_blk[:, None] * n_full[None, :])
        col = jax.lax.broadcasted_iota(jnp.int32, (_BLK, _NP), 1)
        rowg = i * _BLK + jax.lax.broadcasted_iota(jnp.int32, (_BLK, _NP), 0)
        cmask = jnp.where(col < _N, cos, -1e9)
        cur = cmask
        v_k = None
        for _ in range(_K):
            v_k = jnp.max(cur, axis=1, keepdims=True)
            cur = jnp.where(cur == v_k, -2e9, cur)
        selected = jnp.logical_or(cmask >= v_k, col == rowg)
        bias_scr[pl.ds(i * _BLK, _BLK), :] = jnp.where(
            selected, 1.0, 0.0).astype(jnp.float32)

    @pl.when(i >= _NB)
    def _attn_phase():
        emb = emb_ref[...]
        rows = jax.lax.broadcasted_iota(jnp.int32, (_NP, 1), 0)
        valid = rows < _N
        ones_col = jnp.ones((_NP, 1), jnp.float32)
        for k in range(_AB):
            db = data_ref[k]
            xl = jax.lax.dot_general(
                db, lw_ref[...], (((1,), (1,)), ((), ())),
                preferred_element_type=jnp.float32)
            aj_row = (jax.lax.dot_general(
                          attj_ref[...], xl, (((1,), (1,)), ((), ())),
                          preferred_element_type=jnp.float32)
                      + jax.lax.dot_general(
                          attemj_ref[...], emb, (((1,), (1,)), ((), ())),
                          preferred_element_type=jnp.float32))
            a_i = (jax.lax.dot_general(
                       xl, atti_ref[...], (((1,), (1,)), ((), ())),
                       preferred_element_type=jnp.float32)
                   + jax.lax.dot_general(
                       emb, attemi_ref[...], (((1,), (1,)), ((), ())),
                       preferred_element_type=jnp.float32))
            epi = jnp.exp(a_i)
            eni = jnp.exp(0.2 * a_i)
            epj = jnp.exp(aj_row)
            enj = jnp.exp(0.2 * aj_row)
            e = jnp.maximum(epi * epj, eni * enj) * bias_scr[...]
            xl_aug = jnp.concatenate([xl, ones_col], axis=1)
            u = jax.lax.dot_general(
                e, xl_aug, (((1,), (0,)), ((), ())),
                preferred_element_type=jnp.float32)
            inv = 1.0 / (u[:, _D:_D + 1] + 1e-16)
            outv = u[:, :_D] * inv + gb_ref[...]
            out_ref[k] = outv
            ov = jnp.where(valid, outv, 0.0)
            s1_ref[k, 0] = jnp.sum(ov, axis=0)
            s2_ref[k, 0] = jnp.sum(jnp.where(valid, outv * outv, 0.0),
                                   axis=0)


def _bn1_kernel(s1_ref, s2_ref, g1_ref, b1_ref, out_ref, emb_ref,
                sc_ref, sh_ref, t1_ref, t2_ref, st_scr):
    i = pl.program_id(0)

    @pl.when(i == 0)
    def _stats_phase():
        mu = jnp.sum(s1_ref[...], axis=(0, 1)) / _CNT
        var = jnp.sum(s2_ref[...], axis=(0, 1)) / _CNT - mu * mu
        scale = jax.lax.rsqrt(var + _EPS) * g1_ref[0]
        shift = b1_ref[0] - mu * scale
        st_scr[0, :] = scale
        st_scr[1, :] = shift
        sc_ref[0, :] = scale
        sh_ref[0, :] = shift

    @pl.when(i > 0)
    def _t_phase():
        emb = emb_ref[...]
        rows = jax.lax.broadcasted_iota(jnp.int32, (_NP, 1), 0)
        valid = rows < _N
        for k in range(_GB):
            h = out_ref[k] * st_scr[0, :][None, :] + st_scr[1, :][None, :]
            h = jnp.maximum(h, 0.0)
            xo = h * emb
            xv = jnp.where(valid, xo, 0.0)
            t1_ref[k, 0] = jnp.sum(xv, axis=0)
            t2_ref[k, 0] = jnp.sum(jnp.where(valid, xo * xo, 0.0), axis=0)


def _bn2_kernel(t1_ref, t2_ref, g2_ref, b2_ref, sc1_ref, sh1_ref, out_ref,
                emb_ref, ow_ref, ob_ref, pred_ref, st_scr):
    i = pl.program_id(0)

    @pl.when(i == 0)
    def _stats_phase():
        mu = jnp.sum(t1_ref[...], axis=(0, 1)) / _CNT
        var = jnp.sum(t2_ref[...], axis=(0, 1)) / _CNT - mu * mu
        scale = jax.lax.rsqrt(var + _EPS) * g2_ref[0]
        st_scr[0, :] = scale
        st_scr[1, :] = b2_ref[0] - mu * scale

    @pl.when(i > 0)
    def _apply_phase():
        emb = emb_ref[...]
        for k in range(_GB):
            h = out_ref[k] * sc1_ref[...] + sh1_ref[...]
            h = jnp.maximum(h, 0.0)
            xo = h * emb
            y = xo * st_scr[0, :][None, :] + st_scr[1, :][None, :]
            y = jnp.maximum(y, 0.0)
            p = jax.lax.dot_general(
                ow_ref[...], y, (((1,), (1,)), ((), ())),
                preferred_element_type=jnp.float32)
            pred_ref[k] = p + ob_ref[0, 0]


def kernel(data, org_edge_index, embedding_weight, lin_W, att_i, att_j,
           att_em_i, att_em_j, gnn_bias, bn1_gamma, bn1_beta, bn2_gamma,
           bn2_beta, out_W, out_b):
    del org_edge_index
    f32 = jnp.float32
    emb_p = jnp.zeros((_NP, _D), f32).at[:_N].set(embedding_weight)
    data_p = jnp.zeros((_B, _NP, _IN), f32).at[:, :_N].set(data)

    vec = lambda: pl.BlockSpec((1, _D), lambda i: (0, 0))
    out, s1, s2 = pl.pallas_call(
        _attn_kernel,
        grid=(_NB + _B // _AB,),
        in_specs=[
            pl.BlockSpec((_NP, _D), lambda i: (0, 0)),
            pl.BlockSpec((_BLK, _D), lambda i: (jnp.minimum(i, _NB - 1), 0)),
            pl.BlockSpec((_AB, _NP, _IN),
                         lambda i: (jnp.maximum(i - _NB, 0), 0, 0)),
            pl.BlockSpec((_D, _IN), lambda i: (0, 0)),
            vec(), vec(), vec(), vec(), vec(),
        ],
        out_specs=[
            pl.BlockSpec((_AB, _NP, _D), lambda i: (jnp.maximum(i - _NB, 0),
                                                    0, 0)),
            pl.BlockSpec((_AB, 1, _D), lambda i: (jnp.maximum(i - _NB, 0),
                                                  0, 0)),
            pl.BlockSpec((_AB, 1, _D), lambda i: (jnp.maximum(i - _NB, 0),
                                                  0, 0)),
        ],
        out_shape=[
            jax.ShapeDtypeStruct((_B, _NP, _D), f32),
            jax.ShapeDtypeStruct((_B, 1, _D), f32),
            jax.ShapeDtypeStruct((_B, 1, _D), f32),
        ],
        scratch_shapes=[pltpu.VMEM((_NP, _NP), f32)],
    )(emb_p, emb_p, data_p, lin_W, att_i.reshape(1, _D), att_j.reshape(1, _D),
      att_em_i.reshape(1, _D), att_em_j.reshape(1, _D),
      gnn_bias.reshape(1, _D))

    sc1, sh1, t1, t2 = pl.pallas_call(
        _bn1_kernel,
        grid=(1 + _B // _GB,),
        in_specs=[
            pl.BlockSpec((_B, 1, _D), lambda i: (0, 0, 0)),
            pl.BlockSpec((_B, 1, _D), lambda i: (0, 0, 0)),
            pl.BlockSpec((1, _D), lambda i: (0, 0)),
            pl.BlockSpec((1, _D), lambda i: (0, 0)),
            pl.BlockSpec((_GB, _NP, _D), lambda i: (jnp.maximum(i - 1, 0),
                                                    0, 0)),
            pl.BlockSpec((_NP, _D), lambda i: (0, 0)),
        ],
        out_specs=[
            pl.BlockSpec((1, _D), lambda i: (0, 0)),
            pl.BlockSpec((1, _D), lambda i: (0, 0)),
            pl.BlockSpec((_GB, 1, _D), lambda i: (jnp.maximum(i - 1, 0),
                                                  0, 0)),
            pl.BlockSpec((_GB, 1, _D), lambda i: (jnp.maximum(i - 1, 0),
                                                  0, 0)),
        ],
        out_shape=[
            jax.ShapeDtypeStruct((1, _D), f32),
            jax.ShapeDtypeStruct((1, _D), f32),
            jax.ShapeDtypeStruct((_B, 1, _D), f32),
            jax.ShapeDtypeStruct((_B, 1, _D), f32),
        ],
        scratch_shapes=[pltpu.VMEM((2, _D), f32)],
    )(s1, s2, bn1_gamma.reshape(1, _D), bn1_beta.reshape(1, _D), out, emb_p)

    pred = pl.pallas_call(
        _bn2_kernel,
        grid=(1 + _B // _GB,),
        in_specs=[
            pl.BlockSpec((_B, 1, _D), lambda i: (0, 0, 0)),
            pl.BlockSpec((_B, 1, _D), lambda i: (0, 0, 0)),
            pl.BlockSpec((1, _D), lambda i: (0, 0)),
            pl.BlockSpec((1, _D), lambda i: (0, 0)),
            pl.BlockSpec((1, _D), lambda i: (0, 0)),
            pl.BlockSpec((1, _D), lambda i: (0, 0)),
            pl.BlockSpec((_GB, _NP, _D), lambda i: (jnp.maximum(i - 1, 0),
                                                    0, 0)),
            pl.BlockSpec((_NP, _D), lambda i: (0, 0)),
            pl.BlockSpec((1, _D), lambda i: (0, 0)),
            pl.BlockSpec((1, 1), lambda i: (0, 0)),
        ],
        out_specs=pl.BlockSpec((_GB, 1, _NP),
                               lambda i: (jnp.maximum(i - 1, 0), 0, 0)),
        out_shape=jax.ShapeDtypeStruct((_B, 1, _NP), f32),
        scratch_shapes=[pltpu.VMEM((2, _D), f32)],
    )(t1, t2, bn2_gamma.reshape(1, _D), bn2_beta.reshape(1, _D), sc1, sh1,
      out, emb_p, out_W.reshape(1, _D), out_b.reshape(1, 1))

    return pred.reshape(_B, _NP)[:, :_N]


# speedup vs baseline: 1.0098x; 1.0098x over previous
import jax
import jax.numpy as jnp
from jax.experimental import pallas as pl
from jax.experimental.pallas import tpu as pltpu

_N = 1000
_NP = 1024
_D = 64
_IN = 16
_K = 20
_B = 32
_BLK = 128
_NB = _NP // _BLK
_AB = 4
_GB = 8
_CNT = float(_B * _N)
_EPS = 1e-5


def _attn_kernel(emb_ref, embb_ref, data_ref, lw_ref, atti_ref, attj_ref,
                 attemi_ref, attemj_ref, gb_ref, out_ref, s1_ref, s2_ref,
                 bias_scr):
    i = pl.program_id(0)

    @pl.when(i < _NB)
    def _mask_phase():
        w = emb_ref[...]
        wb = embb_ref[...]
        n_full = jnp.sqrt(jnp.maximum(jnp.sum(w * w, axis=1), 1e-12))
        n_blk = jnp.sqrt(jnp.maximum(jnp.sum(wb * wb, axis=1), 1e-12))
        cos = jax.lax.dot_general(
            wb, w, (((1,), (1,)), ((), ())),
            preferred_element_type=jnp.float32)
        cos = cos / (n_blk[:, None] * n_full[None, :])
        col = jax.lax.broadcasted_iota(jnp.int32, (_BLK, _NP), 1)
        rowg = i * _BLK + jax.lax.broadcasted_iota(jnp.int32, (_BLK, _NP), 0)
        cmask = jnp.where(col < _N, cos, -1e9)
        cur = cmask
        v_k = None
        for _ in range(_K):
            v_k = jnp.max(cur, axis=1, keepdims=True)
            cur = jnp.where(cur == v_k, -2e9, cur)
        selected = jnp.logical_or(cmask >= v_k, col == rowg)
        bias_scr[pl.ds(i * _BLK, _BLK), :] = jnp.where(
            selected, 1.0, 0.0).astype(jnp.float32)

    @pl.when(i >= _NB)
    def _attn_phase():
        emb = emb_ref[...]
        rows = jax.lax.broadcasted_iota(jnp.int32, (_NP, 1), 0)
        valid = rows < _N
        ones_col = jnp.ones((_NP, 1), jnp.float32)
        for k in range(_AB):
            db = data_ref[k]
            xl = jax.lax.dot_general(
                db, lw_ref[...], (((1,), (1,)), ((), ())),
                preferred_element_type=jnp.float32)
            aj_row = (jax.lax.dot_general(
                          attj_ref[...], xl, (((1,), (1,)), ((), ())),
                          preferred_element_type=jnp.float32)
                      + jax.lax.dot_general(
                          attemj_ref[...], emb, (((1,), (1,)), ((), ())),
                          preferred_element_type=jnp.float32))
            a_i = (jax.lax.dot_general(
                       xl, atti_ref[...], (((1,), (1,)), ((), ())),
                       preferred_element_type=jnp.float32)
                   + jax.lax.dot_general(
                       emb, attemi_ref[...], (((1,), (1,)), ((), ())),
                       preferred_element_type=jnp.float32))
            epi = jnp.exp(a_i)
            eni = jnp.exp(0.2 * a_i)
            epj = jnp.exp(aj_row)
            enj = jnp.exp(0.2 * aj_row)
            e = jnp.maximum(epi * epj, eni * enj) * bias_scr[...]
            xl_aug = jnp.concatenate([xl, ones_col], axis=1)
            u = jax.lax.dot_general(
                e, xl_aug, (((1,), (0,)), ((), ())),
                preferred_element_type=jnp.float32)
            inv = 1.0 / (u[:, _D:_D + 1] + 1e-16)
            outv = u[:, :_D] * inv + gb_ref[...]
            out_ref[k] = outv
            ov = jnp.where(valid, outv, 0.0)
            s1_ref[k, 0] = jnp.sum(ov, axis=0)
            s2_ref[k, 0] = jnp.sum(jnp.where(valid, outv * outv, 0.0),
                                   axis=0)


def _bn1_kernel(s1_ref, s2_ref, g1_ref, b1_ref, out_ref, emb_ref,
                sc_ref, sh_ref, t1_ref, t2_ref, st_scr):
    i = pl.program_id(0)

    @pl.when(i == 0)
    def _stats_phase():
        mu = jnp.sum(s1_ref[...], axis=(0, 1)) / _CNT
        var = jnp.sum(s2_ref[...], axis=(0, 1)) / _CNT - mu * mu
        scale = jax.lax.rsqrt(var + _EPS) * g1_ref[0]
        shift = b1_ref[0] - mu * scale
        st_scr[0, :] = scale
        st_scr[1, :] = shift
        sc_ref[0, :] = scale
        sh_ref[0, :] = shift

    @pl.when(i > 0)
    def _t_phase():
        emb = emb_ref[...]
        rows = jax.lax.broadcasted_iota(jnp.int32, (_NP, 1), 0)
        valid = rows < _N
        for k in range(_GB):
            h = out_ref[k] * st_scr[0, :][None, :] + st_scr[1, :][None, :]
            h = jnp.maximum(h, 0.0)
            xo = h * emb
            xv = jnp.where(valid, xo, 0.0)
            t1_ref[k, 0] = jnp.sum(xv, axis=0)
            t2_ref[k, 0] = jnp.sum(jnp.where(valid, xo * xo, 0.0), axis=0)


def _bn2_kernel(t1_ref, t2_ref, g2_ref, b2_ref, sc1_ref, sh1_ref, out_ref,
                emb_ref, ow_ref, ob_ref, pred_ref, st_scr):
    i = pl.program_id(0)

    @pl.when(i == 0)
    def _stats_phase():
        mu = jnp.sum(t1_ref[...], axis=(0, 1)) / _CNT
        var = jnp.sum(t2_ref[...], axis=(0, 1)) / _CNT - mu * mu
        scale = jax.lax.rsqrt(var + _EPS) * g2_ref[0]
        st_scr[0, :] = scale
        st_scr[1, :] = b2_ref[0] - mu * scale

    @pl.when(i > 0)
    def _apply_phase():
        emb = emb_ref[...]
        for k in range(_GB):
            h = out_ref[k] * sc1_ref[...] + sh1_ref[...]
            h = jnp.maximum(h, 0.0)
            xo = h * emb
            y = xo * st_scr[0, :][None, :] + st_scr[1, :][None, :]
            y = jnp.maximum(y, 0.0)
            p = jax.lax.dot_general(
                ow_ref[...], y, (((1,), (1,)), ((), ())),
                preferred_element_type=jnp.float32)
            pred_ref[k] = p + ob_ref[0, 0]


def kernel(data, org_edge_index, embedding_weight, lin_W, att_i, att_j,
           att_em_i, att_em_j, gnn_bias, bn1_gamma, bn1_beta, bn2_gamma,
           bn2_beta, out_W, out_b):
    del org_edge_index
    f32 = jnp.float32
    emb_p = jnp.zeros((_NP, _D), f32).at[:_N].set(embedding_weight)
    data_p = jnp.zeros((_B, _NP, _IN), f32).at[:, :_N].set(data)

    vec = lambda: pl.BlockSpec((1, _D), lambda i: (0, 0))
    out, s1, s2 = pl.pallas_call(
        _attn_kernel,
        grid=(_NB + _B // _AB,),
        in_specs=[
            pl.BlockSpec((_NP, _D), lambda i: (0, 0)),
            pl.BlockSpec((_BLK, _D), lambda i: (jnp.minimum(i, _NB - 1), 0)),
            pl.BlockSpec((_AB, _NP, _IN),
                         lambda i: (jnp.maximum(i - _NB, 0), 0, 0)),
            pl.BlockSpec((_D, _IN), lambda i: (0, 0)),
            vec(), vec(), vec(), vec(), vec(),
        ],
        out_specs=[
            pl.BlockSpec((_AB, _NP, _D), lambda i: (jnp.maximum(i - _NB, 0),
                                                    0, 0)),
            pl.BlockSpec((_AB, 1, _D), lambda i: (jnp.maximum(i - _NB, 0),
                                                  0, 0)),
            pl.BlockSpec((_AB, 1, _D), lambda i: (jnp.maximum(i - _NB, 0),
                                                  0, 0)),
        ],
        out_shape=[
            jax.ShapeDtypeStruct((_B, _NP, _D), f32),
            jax.ShapeDtypeStruct((_B, 1, _D), f32),
            jax.ShapeDtypeStruct((_B, 1, _D), f32),
        ],
        scratch_shapes=[pltpu.VMEM((_NP, _NP), f32)],
    )(emb_p, emb_p, data_p, lin_W, att_i.reshape(1, _D), att_j.reshape(1, _D),
      att_em_i.reshape(1, _D), att_em_j.reshape(1, _D),
      gnn_bias.reshape(1, _D))

    sc1, sh1, t1, t2 = pl.pallas_call(
        _bn1_kernel,
        grid=(1 + _B // _GB,),
        in_specs=[
            pl.BlockSpec((_B, 1, _D), lambda i: (0, 0, 0)),
            pl.BlockSpec((_B, 1, _D), lambda i: (0, 0, 0)),
            pl.BlockSpec((1, _D), lambda i: (0, 0)),
            pl.BlockSpec((1, _D), lambda i: (0, 0)),
            pl.BlockSpec((_GB, _NP, _D), lambda i: (jnp.maximum(i - 1, 0),
                                                    0, 0)),
            pl.BlockSpec((_NP, _D), lambda i: (0, 0)),
        ],
        out_specs=[
            pl.BlockSpec((1, _D), lambda i: (0, 0)),
            pl.BlockSpec((1, _D), lambda i: (0, 0)),
            pl.BlockSpec((_GB, 1, _D), lambda i: (jnp.maximum(i - 1, 0),
                                                  0, 0)),
            pl.BlockSpec((_GB, 1, _D), lambda i: (jnp.maximum(i - 1, 0),
                                                  0, 0)),
        ],
        out_shape=[
            jax.ShapeDtypeStruct((1, _D), f32),
            jax.ShapeDtypeStruct((1, _D), f32),
            jax.ShapeDtypeStruct((_B, 1, _D), f32),
            jax.ShapeDtypeStruct((_B, 1, _D), f32),
        ],
        scratch_shapes=[pltpu.VMEM((2, _D), f32)],
    )(s1, s2, bn1_gamma.reshape(1, _D), bn1_beta.reshape(1, _D), out, emb_p)

    pred = pl.pallas_call(
        _bn2_kernel,
        grid=(1 + _B // _GB,),
        in_specs=[
            pl.BlockSpec((_B, 1, _D), lambda i: (0, 0, 0)),
            pl.BlockSpec((_B, 1, _D), lambda i: (0, 0, 0)),
            pl.BlockSpec((1, _D), lambda i: (0, 0)),
            pl.BlockSpec((1, _D), lambda i: (0, 0)),
            pl.BlockSpec((1, _D), lambda i: (0, 0)),
            pl.BlockSpec((1, _D), lambda i: (0, 0)),
            pl.BlockSpec((_GB, _NP, _D), lambda i: (jnp.maximum(i - 1, 0),
                                                    0, 0)),
            pl.BlockSpec((_NP, _D), lambda i: (0, 0)),
            pl.BlockSpec((1, _D), lambda i: (0, 0)),
            pl.BlockSpec((1, 1), lambda i: (0, 0)),
        ],
        out_specs=pl.BlockSpec((_GB, 1, _NP),
                               lambda i: (jnp.maximum(i - 1, 0), 0, 0)),
        out_shape=jax.ShapeDtypeStruct((_B, 1, _NP), f32),
        scratch_shapes=[pltpu.VMEM((2, _D), f32)],
    )(t1, t2, bn2_gamma.reshape(1, _D), bn2_beta.reshape(1, _D), sc1, sh1,
      out, emb_p, out_W.reshape(1, _D), out_b.reshape(1, 1))

    return pred.reshape(_B, _NP)[:, :_N]
